# baseline (speedup 1.0000x reference)
import jax
import jax.numpy as jnp
from jax.experimental import pallas as pl
from jax.experimental.pallas import tpu as pltpu

R = 128
C = 100000
RB = 8
CHUNK = 256
NFULL = C // CHUNK
CPAD = (NFULL + 1) * CHUNK
NCHUNK = NFULL + 1
EPS = 1e-10
NEG_INF = float("-inf")


def _threefry_bits(x1):
    u32 = jnp.uint32
    ks0 = u32(0)
    ks1 = u32(1)
    ks2 = u32(0x1BD11BDA) ^ ks0 ^ ks1

    def rotl(x, r):
        return (x << u32(r)) | (x >> u32(32 - r))

    x0 = x1
    x1 = rotl(x1, 13) ^ x0
    rots = ((13, 15, 26, 6), (17, 29, 16, 24))
    adds = ((ks1, ks2 + u32(1)), (ks2, ks0 + u32(2)), (ks0, ks1 + u32(3)),
            (ks1, ks2 + u32(4)), (ks2, ks0 + u32(5)))
    for g in range(5):
        for r in rots[g % 2][1 if g == 0 else 0:]:
            x0 = x0 + x1
            x1 = rotl(x1, r)
            x1 = x1 ^ x0
        x0 = x0 + adds[g][0]
        x1 = x1 + adds[g][1]
    return x0 ^ x1


def _fused_kernel(x_ref, o_ref):
    i = pl.program_id(0)
    col0 = jax.lax.broadcasted_iota(jnp.int32, (RB, CHUNK), 1)
    rowc = (jax.lax.broadcasted_iota(jnp.int32, (RB, 1), 0) + i * RB) * C
    base1 = (rowc + col0 + 1).astype(jnp.uint32)

    def chunk_z(c):
        bits = _threefry_bits(base1 + jnp.asarray(c * CHUNK).astype(jnp.uint32))
        fb = (bits >> jnp.uint32(9)) | jnp.uint32(0x3F800000)
        u = jax.lax.bitcast_convert_type(fb, jnp.float32) - jnp.float32(1.0)
        w = jnp.float32(EPS) - jnp.log(u + jnp.float32(EPS))
        return x_ref[:, pl.ds(c * CHUNK, CHUNK)] - jnp.log(w)

    def scan_body(c, carry):
        m, bi = carry
        z = chunk_z(c)
        better = z > m
        return jnp.maximum(z, m), jnp.where(better, c, bi)

    init = (jnp.full((RB, CHUNK), NEG_INF, jnp.float32),
            jnp.zeros((RB, CHUNK), jnp.int32))
    m, bi = jax.lax.fori_loop(0, NFULL, scan_body, init, unroll=16)

    z_t = jnp.where(col0 + NFULL * CHUNK < C, chunk_z(NFULL), NEG_INF)
    better_t = z_t > m
    m = jnp.maximum(z_t, m)
    bi = jnp.where(better_t, NFULL, bi)

    col = bi * CHUNK + col0
    best = jnp.max(m, axis=1, keepdims=True)
    idx = jnp.min(jnp.where(m == best, col, jnp.int32(C)), axis=1,
                  keepdims=True)

    def write_body(c, _):
        o_ref[:, pl.ds(c * CHUNK, CHUNK)] = jnp.where(
            col0 + c * CHUNK == idx, jnp.float32(1.0), jnp.float32(0.0))
        return 0

    jax.lax.fori_loop(0, NCHUNK, write_body, 0, unroll=8)


def kernel(logits):
    return pl.pallas_call(
        _fused_kernel,
        grid=(R // RB,),
        in_specs=[pl.BlockSpec((RB, CPAD), lambda i: (i, 0))],
        out_specs=pl.BlockSpec((RB, CPAD), lambda i: (i, 0)),
        out_shape=jax.ShapeDtypeStruct((R, C), jnp.float32),
        compiler_params=pltpu.CompilerParams(
            dimension_semantics=("arbitrary",)),
    )(logits)


# speedup vs baseline: 1.0143x; 1.0143x over previous
import jax
import jax.numpy as jnp
from jax.experimental import pallas as pl
from jax.experimental.pallas import tpu as pltpu

R = 128
C = 100000
RB = 16
CHUNK = 256
NFULL = C // CHUNK
CPAD = (NFULL + 1) * CHUNK
NCHUNK = NFULL + 1
EPS = 1e-10
NEG_INF = float("-inf")


def _threefry_bits(x1):
    u32 = jnp.uint32
    ks0 = u32(0)
    ks1 = u32(1)
    ks2 = u32(0x1BD11BDA) ^ ks0 ^ ks1

    def rotl(x, r):
        return (x << u32(r)) | (x >> u32(32 - r))

    x0 = x1
    x1 = rotl(x1, 13) ^ x0
    rots = ((13, 15, 26, 6), (17, 29, 16, 24))
    adds = ((ks1, ks2 + u32(1)), (ks2, ks0 + u32(2)), (ks0, ks1 + u32(3)),
            (ks1, ks2 + u32(4)), (ks2, ks0 + u32(5)))
    for g in range(5):
        for r in rots[g % 2][1 if g == 0 else 0:]:
            x0 = x0 + x1
            x1 = rotl(x1, r)
            x1 = x1 ^ x0
        x0 = x0 + adds[g][0]
        x1 = x1 + adds[g][1]
    return x0 ^ x1


def _fused_kernel(x_ref, o_ref):
    i = pl.program_id(0)
    col0 = jax.lax.broadcasted_iota(jnp.int32, (RB, CHUNK), 1)
    rowc = (jax.lax.broadcasted_iota(jnp.int32, (RB, 1), 0) + i * RB) * C
    base1 = (rowc + col0 + 1).astype(jnp.uint32)

    def chunk_z(c):
        bits = _threefry_bits(base1 + jnp.asarray(c * CHUNK).astype(jnp.uint32))
        fb = (bits >> jnp.uint32(9)) | jnp.uint32(0x3F800000)
        u = jax.lax.bitcast_convert_type(fb, jnp.float32) - jnp.float32(1.0)
        w = jnp.float32(EPS) - jnp.log(u + jnp.float32(EPS))
        return x_ref[:, pl.ds(c * CHUNK, CHUNK)] - jnp.log(w)

    def scan_body(c, carry):
        m, bi = carry
        z = chunk_z(c)
        better = z > m
        return jnp.maximum(z, m), jnp.where(better, c, bi)

    init = (jnp.full((RB, CHUNK), NEG_INF, jnp.float32),
            jnp.zeros((RB, CHUNK), jnp.int32))
    m, bi = jax.lax.fori_loop(0, NFULL, scan_body, init, unroll=13)

    z_t = jnp.where(col0 + NFULL * CHUNK < C, chunk_z(NFULL), NEG_INF)
    better_t = z_t > m
    m = jnp.maximum(z_t, m)
    bi = jnp.where(better_t, NFULL, bi)

    col = bi * CHUNK + col0
    best = jnp.max(m, axis=1, keepdims=True)
    idx = jnp.min(jnp.where(m == best, col, jnp.int32(C)), axis=1,
                  keepdims=True)

    def write_body(c, _):
        o_ref[:, pl.ds(c * CHUNK, CHUNK)] = jnp.where(
            col0 + c * CHUNK == idx, jnp.float32(1.0), jnp.float32(0.0))
        return 0

    jax.lax.fori_loop(0, NCHUNK, write_body, 0, unroll=8)


def kernel(logits):
    return pl.pallas_call(
        _fused_kernel,
        grid=(R // RB,),
        in_specs=[pl.BlockSpec((RB, CPAD), lambda i: (i, 0))],
        out_specs=pl.BlockSpec((RB, CPAD), lambda i: (i, 0)),
        out_shape=jax.ShapeDtypeStruct((R, C), jnp.float32),
        compiler_params=pltpu.CompilerParams(
            dimension_semantics=("arbitrary",)),
    )(logits)


# speedup vs baseline: 1.0241x; 1.0096x over previous
import jax
import jax.numpy as jnp
from jax.experimental import pallas as pl
from jax.experimental.pallas import tpu as pltpu

R = 128
C = 100000
RB = 16
CHUNK = 512
NFULL = C // CHUNK
CPAD = (NFULL + 1) * CHUNK
NCHUNK = NFULL + 1
EPS = 1e-10
NEG_INF = float("-inf")


def _threefry_bits(x1):
    u32 = jnp.uint32
    ks0 = u32(0)
    ks1 = u32(1)
    ks2 = u32(0x1BD11BDA) ^ ks0 ^ ks1

    def rotl(x, r):
        return (x << u32(r)) | (x >> u32(32 - r))

    x0 = x1
    x1 = rotl(x1, 13) ^ x0
    rots = ((13, 15, 26, 6), (17, 29, 16, 24))
    adds = ((ks1, ks2 + u32(1)), (ks2, ks0 + u32(2)), (ks0, ks1 + u32(3)),
            (ks1, ks2 + u32(4)), (ks2, ks0 + u32(5)))
    for g in range(5):
        for r in rots[g % 2][1 if g == 0 else 0:]:
            x0 = x0 + x1
            x1 = rotl(x1, r)
            x1 = x1 ^ x0
        x0 = x0 + adds[g][0]
        x1 = x1 + adds[g][1]
    return x0 ^ x1


def _fused_kernel(x_ref, o_ref):
    i = pl.program_id(0)
    col0 = jax.lax.broadcasted_iota(jnp.int32, (RB, CHUNK), 1)
    rowc = (jax.lax.broadcasted_iota(jnp.int32, (RB, 1), 0) + i * RB) * C
    base1 = (rowc + col0 + 1).astype(jnp.uint32)

    def chunk_z(c):
        bits = _threefry_bits(base1 + jnp.asarray(c * CHUNK).astype(jnp.uint32))
        fb = (bits >> jnp.uint32(9)) | jnp.uint32(0x3F800000)
        u = jax.lax.bitcast_convert_type(fb, jnp.float32) - jnp.float32(1.0)
        w = jnp.float32(EPS) - jnp.log(u + jnp.float32(EPS))
        return x_ref[:, pl.ds(c * CHUNK, CHUNK)] - jnp.log(w)

    def scan_body(c, carry):
        m, bi = carry
        z = chunk_z(c)
        better = z > m
        return jnp.maximum(z, m), jnp.where(better, c, bi)

    init = (jnp.full((RB, CHUNK), NEG_INF, jnp.float32),
            jnp.zeros((RB, CHUNK), jnp.int32))
    m, bi = jax.lax.fori_loop(0, NFULL, scan_body, init, unroll=13)

    z_t = jnp.where(col0 + NFULL * CHUNK < C, chunk_z(NFULL), NEG_INF)
    better_t = z_t > m
    m = jnp.maximum(z_t, m)
    bi = jnp.where(better_t, NFULL, bi)

    col = bi * CHUNK + col0
    best = jnp.max(m, axis=1, keepdims=True)
    idx = jnp.min(jnp.where(m == best, col, jnp.int32(C)), axis=1,
                  keepdims=True)

    def write_body(c, _):
        o_ref[:, pl.ds(c * CHUNK, CHUNK)] = jnp.where(
            col0 + c * CHUNK == idx, jnp.float32(1.0), jnp.float32(0.0))
        return 0

    jax.lax.fori_loop(0, NCHUNK, write_body, 0, unroll=8)


def kernel(logits):
    return pl.pallas_call(
        _fused_kernel,
        grid=(R // RB,),
        in_specs=[pl.BlockSpec((RB, CPAD), lambda i: (i, 0))],
        out_specs=pl.BlockSpec((RB, CPAD), lambda i: (i, 0)),
        out_shape=jax.ShapeDtypeStruct((R, C), jnp.float32),
        compiler_params=pltpu.CompilerParams(
            dimension_semantics=("arbitrary",)),
    )(logits)
